# Initial kernel scaffold; baseline (speedup 1.0000x reference)
import jax
import jax.numpy as jnp
from jax.experimental import pallas as pl


def kernel(word_ids, cat1_ids, cat2_ids, cat3_ids, numerics, word_emb, cat1_emb, cat2_emb, cat3_emb, t_proj_w, t_proj_b, t_ln_g, t_ln_b, n_proj_w, n_proj_b, n_ln_g, n_ln_b, f1_w, f1_b, f_ln_g, f_ln_b, f2_w, f2_b):
    raise NotImplementedError("write your pallas kernel here")


# speedup vs baseline: 3.2182x; 3.2182x over previous
import functools

import jax
import jax.numpy as jnp
from jax import lax
from jax.experimental import pallas as pl
from jax.experimental.pallas import tpu as pltpu
from jax.experimental.pallas import tpu_sc as plsc

_B = 16384
_L = 20
_NW = 32
_ROWS_W = _B // _NW
_GROUP = 32
_NGROUP = _ROWS_W // _GROUP
_TOK = _GROUP * _L
_NCH = _TOK // 128
_CCH = _ROWS_W // 128


def _sc_body(wid3, c1i, c2i, c3i, wemb, c1e, c2e, c3e,
             pooled_out, c1_out, c2_out, c3_out,
             idx_v, rows_v, out_v, cidx_v, crows_v, sem):
    w = lax.axis_index("c") * 16 + lax.axis_index("s")

    for ids_h, emb_h, out_h in ((c1i, c1e, c1_out),
                                (c2i, c2e, c2_out),
                                (c3i, c3e, c3_out)):
        pltpu.sync_copy(ids_h.at[w], cidx_v)
        cps = [pltpu.async_copy(emb_h.at[cidx_v.at[j]],
                                crows_v.at[pl.ds(j * 128, 128)], sem)
               for j in range(_CCH)]
        for cp in cps:
            cp.wait()
        pltpu.sync_copy(crows_v, out_h.at[pl.ds(w * _ROWS_W, _ROWS_W)])

    def gbody(g, carry):
        pltpu.sync_copy(wid3.at[w * _NGROUP + g], idx_v)
        cps = [pltpu.async_copy(wemb.at[idx_v.at[j]],
                                rows_v.at[pl.ds(j * 128, 128)], sem)
               for j in range(_NCH)]
        for cp in cps:
            cp.wait()

        def rbody(r, c2):
            base = r * _L
            acc = [rows_v[base, pl.ds(f * 16, 16)] for f in range(4)]
            for l in range(1, _L):
                for f in range(4):
                    acc[f] = acc[f] + rows_v[base + l, pl.ds(f * 16, 16)]
            for f in range(4):
                out_v[r, pl.ds(f * 16, 16)] = acc[f]
            return c2

        lax.fori_loop(0, _GROUP, rbody, 0)
        pltpu.sync_copy(out_v,
                        pooled_out.at[pl.ds(w * _ROWS_W + g * _GROUP, _GROUP)])
        return carry

    lax.fori_loop(0, _NGROUP, gbody, 0)


_sc_gather = functools.partial(
    pl.kernel,
    out_type=[
        jax.ShapeDtypeStruct((_B, 64), jnp.float32),
        jax.ShapeDtypeStruct((_B, 16), jnp.float32),
        jax.ShapeDtypeStruct((_B, 16), jnp.float32),
        jax.ShapeDtypeStruct((_B, 16), jnp.float32),
    ],
    mesh=plsc.VectorSubcoreMesh(core_axis_name="c", subcore_axis_name="s"),
    compiler_params=pltpu.CompilerParams(use_tc_tiling_on_sc=False),
    scratch_types=[
        pltpu.VMEM((_NCH, 128), jnp.int32),
        pltpu.VMEM((_TOK, 64), jnp.float32),
        pltpu.VMEM((_GROUP, 64), jnp.float32),
        pltpu.VMEM((_CCH, 128), jnp.int32),
        pltpu.VMEM((_ROWS_W, 16), jnp.float32),
        pltpu.SemaphoreType.DMA,
    ],
)(_sc_body)


def _ln(x, g, b, eps=1e-5):
    m = jnp.mean(x, axis=-1, keepdims=True)
    v = jnp.mean((x - m) ** 2, axis=-1, keepdims=True)
    return (x - m) / jnp.sqrt(v + eps) * g + b


def _gelu(x):
    return 0.5 * x * (1.0 + lax.erf(x * 0.7071067811865476))


_BLK = 1024


def _tc_body(wids_ref, pooled_ref, c1_ref, c2_ref, c3_ref, num_ref,
             tpw, tpb, tlg, tlb, npw, npb, nlg, nlb,
             f1w, f1b, flg, flb, f2w, f2b, out_ref):
    cnt = jnp.sum((wids_ref[...] != 0).astype(jnp.float32), axis=1,
                  keepdims=True)
    mean = pooled_ref[...] / jnp.maximum(cnt, 1.0)
    t = _gelu(_ln(jnp.dot(mean, tpw[...],
                          preferred_element_type=jnp.float32) + tpb[...],
                  tlg[...], tlb[...]))
    n = _gelu(_ln(jnp.dot(num_ref[...], npw[...],
                          preferred_element_type=jnp.float32) + npb[...],
                  nlg[...], nlb[...]))
    fused = jnp.concatenate([t, c1_ref[...], c2_ref[...], c3_ref[...], n],
                            axis=-1)
    h = _gelu(_ln(jnp.dot(fused, f1w[...],
                          preferred_element_type=jnp.float32) + f1b[...],
                  flg[...], flb[...]))
    out_ref[...] = jnp.dot(h, f2w[...],
                           preferred_element_type=jnp.float32) + f2b[...]


def _full(shape):
    return pl.BlockSpec(shape, lambda i: (0,) * len(shape))


_tc_encode = pl.pallas_call(
    _tc_body,
    grid=(_B // _BLK,),
    in_specs=[
        pl.BlockSpec((_BLK, _L), lambda i: (i, 0)),
        pl.BlockSpec((_BLK, 64), lambda i: (i, 0)),
        pl.BlockSpec((_BLK, 16), lambda i: (i, 0)),
        pl.BlockSpec((_BLK, 16), lambda i: (i, 0)),
        pl.BlockSpec((_BLK, 16), lambda i: (i, 0)),
        pl.BlockSpec((_BLK, 2), lambda i: (i, 0)),
        _full((64, 64)), _full((64,)), _full((64,)), _full((64,)),
        _full((2, 16)), _full((16,)), _full((16,)), _full((16,)),
        _full((128, 128)), _full((128,)), _full((128,)), _full((128,)),
        _full((128, 128)), _full((128,)),
    ],
    out_specs=pl.BlockSpec((_BLK, 128), lambda i: (i, 0)),
    out_shape=jax.ShapeDtypeStruct((_B, 128), jnp.float32),
)


def kernel(word_ids, cat1_ids, cat2_ids, cat3_ids, numerics,
           word_emb, cat1_emb, cat2_emb, cat3_emb,
           t_proj_w, t_proj_b, t_ln_g, t_ln_b,
           n_proj_w, n_proj_b, n_ln_g, n_ln_b,
           f1_w, f1_b, f_ln_g, f_ln_b, f2_w, f2_b):
    wid3 = word_ids.reshape(_NW * _NGROUP, _NCH, 128)
    c1i = cat1_ids.reshape(_NW, _CCH, 128)
    c2i = cat2_ids.reshape(_NW, _CCH, 128)
    c3i = cat3_ids.reshape(_NW, _CCH, 128)
    pooled, c1, c2, c3 = _sc_gather(wid3, c1i, c2i, c3i,
                                    word_emb, cat1_emb, cat2_emb, cat3_emb)
    return _tc_encode(word_ids, pooled, c1, c2, c3, numerics,
                      t_proj_w.T, t_proj_b, t_ln_g, t_ln_b,
                      n_proj_w.T, n_proj_b, n_ln_g, n_ln_b,
                      f1_w.T, f1_b, f_ln_g, f_ln_b,
                      f2_w.T, f2_b)
